# baseline (speedup 1.0000x reference)
import functools

import jax
import jax.numpy as jnp
from jax import lax
from jax.experimental import pallas as pl
from jax.experimental.pallas import tpu as pltpu
from jax.experimental.pallas import tpu_sc as plsc

X_MIN, X_MAX = -50.0, 50.0
Y_MIN, Y_MAX = -50.0, 50.0
RES = 0.25
BEV_W = int((X_MAX - X_MIN) / RES)
BEV_H = int((Y_MAX - Y_MIN) / RES)
NUM_CELLS = BEV_W * BEV_H
NUM_CLASSES = 10
BOX_DIM = 7
BSZ = 4
NB = 128

_NC, _NS = 2, 16
_NW = _NC * _NS
_CHUNKS = NB // 16
assert BSZ * _CHUNKS == _NW


def _softplus_sum_body(x_ref, o_ref):
    x = x_ref[...]
    ax = jnp.abs(x)
    s = jnp.sum(0.5 * (x + ax) + jnp.log(1.0 + jnp.exp(-ax)))

    @pl.when(pl.program_id(0) == 0)
    def _():
        o_ref[0, 0] = 0.0

    o_ref[0, 0] += s


def _dense_softplus_sum(cls_t):
    out = pl.pallas_call(
        _softplus_sum_body,
        grid=(50,),
        in_specs=[pl.BlockSpec((NUM_CLASSES, BSZ, 3200), lambda i: (0, 0, i))],
        out_specs=pl.BlockSpec(memory_space=pltpu.SMEM),
        out_shape=jax.ShapeDtypeStruct((1, 1), jnp.float32),
    )(cls_t)
    return out[0, 0]


def _dyn_gather16(vec, idx):
    dn = lax.GatherDimensionNumbers(offset_dims=(), collapsed_slice_dims=(0,),
                                    start_index_map=(0,))
    return lax.gather(vec, idx[:, None], dn, (1,),
                      mode=lax.GatherScatterMode.PROMISE_IN_BOUNDS)


def _sc_body(cls_hbm, boxp_hbm, gtb_hbm, lab_hbm, msk_hbm, out_hbm,
             gtb_v, lab_v, msk_v, idx_v, cwin_v, bwin_v, stage_v,
             sem):
    cid = lax.axis_index("c")
    sid = lax.axis_index("s")
    wid = sid * _NC + cid
    b = wid // _CHUNKS
    jc = wid % _CHUNKS
    jlo = jc * 16

    pltpu.sync_copy(gtb_hbm, gtb_v)
    pltpu.sync_copy(lab_hbm, lab_v)
    pltpu.sync_copy(msk_hbm, msk_v)

    iota16 = lax.iota(jnp.int32, 16)

    for c in range(_CHUNKS):
        sl = pl.ds(c * 16, 16)
        x = gtb_v[0, b, sl]
        y = gtb_v[1, b, sl]
        m = msk_v[b, sl]
        lb = lab_v[b, sl]
        valid = ((m > 0.5) & (lb >= 0)
                 & (x >= X_MIN) & (x <= X_MAX)
                 & (y >= Y_MIN) & (y <= Y_MAX))
        gx = jnp.clip(((x - X_MIN) / RES).astype(jnp.int32), 0, BEV_W - 1)
        gy = jnp.clip(((y - Y_MIN) / RES).astype(jnp.int32), 0, BEV_H - 1)
        cell = gy * BEV_W + gx
        idx_v[pl.ds(c * 16, 16)] = jnp.where(valid, cell, -1)

    jpos = jlo + iota16
    jidx = idx_v[pl.ds(jlo, 16)]
    jlab = lab_v[b, pl.ds(jlo, 16)]
    safe_cell = jnp.where(jidx >= 0, jidx, 0)
    jlabc = jnp.clip(jlab, 0, NUM_CLASSES - 1)

    one16 = jnp.ones((16,), jnp.int32)
    zero16 = jnp.zeros((16,), jnp.int32)
    dupb = zero16
    dupc = zero16
    for c in range(_CHUNKS):
        kidx = idx_v[pl.ds(c * 16, 16)]
        klab = lab_v[b, pl.ds(c * 16, 16)]
        for i in range(16):
            k = c * 16 + i
            later = jnp.where(k > jpos, one16, zero16)
            same = later * jnp.where(kidx[i] == jidx, one16, zero16)
            dupb = dupb | same
            dupc = dupc | (same * jnp.where(klab[i] == jlab, one16, zero16))

    validm = jnp.where(jidx >= 0, one16, zero16)
    keepc = validm * (1 - dupc)
    winner = validm * (1 - dupb)

    ca = safe_cell & ~127
    descs = []
    for i in range(16):
        cai = pl.multiple_of(ca[i], 128)
        dc = pltpu.make_async_copy(
            cls_hbm.at[jlabc[i], :, pl.ds(cai, 128)], cwin_v.at[i], sem)
        db = pltpu.make_async_copy(
            boxp_hbm.at[:, :, pl.ds(cai, 128)], bwin_v.at[i], sem)
        pc = keepc[i] > 0
        pb = winner[i] > 0

        @pl.when(pc)
        def _(dc=dc):
            dc.start()

        @pl.when(pb)
        def _(db=db):
            db.start()

        descs.append((pc, dc))
        descs.append((pb, db))
    for pred, dsc in descs:
        @pl.when(pred)
        def _(dsc=dsc):
            dsc.wait()

    clow = safe_cell & 127
    zf16 = jnp.zeros((16,), jnp.float32)
    cvals = zf16
    bp = [zf16] * BOX_DIM
    for i in range(16):
        st = pl.multiple_of(clow[i] & ~15, 16)
        r = jnp.zeros((16,), jnp.int32) + (clow[i] & 15)
        csub = cwin_v[i, b, pl.ds(st, 16)]
        ev = _dyn_gather16(csub, r)
        cvals = jnp.where(iota16 == i, ev, cvals)
        for d in range(BOX_DIM):
            bsub = bwin_v[i, d, b, pl.ds(st, 16)]
            evd = _dyn_gather16(bsub, r)
            bp[d] = jnp.where(iota16 == i, evd, bp[d])

    sl1 = jnp.zeros((16,), jnp.float32)
    for d in range(BOX_DIM):
        diff = bp[d] - gtb_v[d, b, pl.ds(jlo, 16)]
        ad = jnp.abs(diff)
        sl1 = sl1 + jnp.where(ad < 1.0, 0.5 * diff * diff, ad - 0.5)

    stage_v[0, :] = jnp.where(keepc > 0, -cvals, 0.0)
    stage_v[1, :] = jnp.where(winner > 0, sl1, 0.0)
    stage_v[2, :] = jnp.where(winner > 0, 1.0, 0.0)
    pltpu.sync_copy(stage_v, out_hbm.at[wid])


def _sc_sparse_part(cls_t, boxp_t, gtb_t, lab, msk):
    mesh = plsc.VectorSubcoreMesh(core_axis_name="c", subcore_axis_name="s",
                                  num_cores=_NC, num_subcores=_NS)
    fn = pl.kernel(
        _sc_body,
        out_type=jax.ShapeDtypeStruct((_NW, 3, 16), jnp.float32),
        mesh=mesh,
        scratch_types=[
            pltpu.VMEM((BOX_DIM, BSZ, NB), jnp.float32),
            pltpu.VMEM((BSZ, NB), jnp.int32),
            pltpu.VMEM((BSZ, NB), jnp.float32),
            pltpu.VMEM((NB,), jnp.int32),
            pltpu.VMEM((16, BSZ, 128), jnp.float32),
            pltpu.VMEM((16, BOX_DIM, BSZ, 128), jnp.float32),
            pltpu.VMEM((3, 16), jnp.float32),
            pltpu.SemaphoreType.DMA,
        ],
    )
    return fn(cls_t, boxp_t, gtb_t, lab, msk)


def kernel(cls_logits, box_preds, gt_boxes, gt_labels, gt_masks):
    bsz = cls_logits.shape[0]
    cls_t = jnp.transpose(cls_logits, (2, 0, 1))
    boxp_t = jnp.transpose(box_preds, (2, 0, 1))
    gtb_t = jnp.transpose(gt_boxes, (2, 0, 1))
    lab = gt_labels.astype(jnp.int32)

    dense_sum = _dense_softplus_sum(cls_t)
    parts = _sc_sparse_part(cls_t, boxp_t, gtb_t, lab, gt_masks)
    sums = parts.sum(axis=(0, 2))

    cls_loss = (dense_sum + sums[0]) / (bsz * NUM_CELLS)
    box_loss = jnp.where(sums[2] > 0, sums[1] / (sums[2] + 1e-6), 0.0)
    total = cls_loss + box_loss
    return total, cls_loss, box_loss


# speedup vs baseline: 1.4005x; 1.4005x over previous
import functools

import jax
import jax.numpy as jnp
from jax import lax
from jax.experimental import pallas as pl
from jax.experimental.pallas import tpu as pltpu
from jax.experimental.pallas import tpu_sc as plsc

X_MIN, X_MAX = -50.0, 50.0
Y_MIN, Y_MAX = -50.0, 50.0
RES = 0.25
BEV_W = int((X_MAX - X_MIN) / RES)
BEV_H = int((Y_MAX - Y_MIN) / RES)
NUM_CELLS = BEV_W * BEV_H
NUM_CLASSES = 10
BOX_DIM = 7
BSZ = 4
NB = 128

_NC, _NS = 2, 16
_NW = _NC * _NS
_CHUNKS = NB // 16
assert BSZ * _CHUNKS == _NW


def _softplus_sum_body(x_ref, o_ref):
    x = x_ref[...]
    ax = jnp.abs(x)
    s = jnp.sum(0.5 * (x + ax) + jnp.log(1.0 + jnp.exp(-ax)))

    @pl.when(pl.program_id(0) == 0)
    def _():
        o_ref[0, 0] = 0.0

    o_ref[0, 0] += s


def _dense_softplus_sum(cls_t):
    out = pl.pallas_call(
        _softplus_sum_body,
        grid=(10,),
        in_specs=[pl.BlockSpec((NUM_CLASSES, BSZ, 16000), lambda i: (0, 0, i))],
        out_specs=pl.BlockSpec(memory_space=pltpu.SMEM),
        out_shape=jax.ShapeDtypeStruct((1, 1), jnp.float32),
    )(cls_t)
    return out[0, 0]


def _dyn_gather16(vec, idx):
    dn = lax.GatherDimensionNumbers(offset_dims=(), collapsed_slice_dims=(0,),
                                    start_index_map=(0,))
    return lax.gather(vec, idx[:, None], dn, (1,),
                      mode=lax.GatherScatterMode.PROMISE_IN_BOUNDS)


def _sc_body(cls_hbm, boxp_hbm, gtb_hbm, lab_hbm, msk_hbm, out_hbm,
             gtb_v, lab_v, msk_v, idx_v, cwin_v, bwin_v, stage_v,
             sem):
    cid = lax.axis_index("c")
    sid = lax.axis_index("s")
    wid = sid * _NC + cid
    b = wid // _CHUNKS
    jc = wid % _CHUNKS
    jlo = jc * 16

    pltpu.sync_copy(gtb_hbm, gtb_v)
    pltpu.sync_copy(lab_hbm, lab_v)
    pltpu.sync_copy(msk_hbm, msk_v)

    iota16 = lax.iota(jnp.int32, 16)

    for c in range(_CHUNKS):
        sl = pl.ds(c * 16, 16)
        x = gtb_v[0, b, sl]
        y = gtb_v[1, b, sl]
        m = msk_v[b, sl]
        lb = lab_v[b, sl]
        valid = ((m > 0.5) & (lb >= 0)
                 & (x >= X_MIN) & (x <= X_MAX)
                 & (y >= Y_MIN) & (y <= Y_MAX))
        gx = jnp.clip(((x - X_MIN) / RES).astype(jnp.int32), 0, BEV_W - 1)
        gy = jnp.clip(((y - Y_MIN) / RES).astype(jnp.int32), 0, BEV_H - 1)
        cell = gy * BEV_W + gx
        idx_v[pl.ds(c * 16, 16)] = jnp.where(valid, cell, -1)

    jpos = jlo + iota16
    jidx = idx_v[pl.ds(jlo, 16)]
    jlab = lab_v[b, pl.ds(jlo, 16)]
    safe_cell = jnp.where(jidx >= 0, jidx, 0)
    jlabc = jnp.clip(jlab, 0, NUM_CLASSES - 1)

    one16 = jnp.ones((16,), jnp.int32)
    zero16 = jnp.zeros((16,), jnp.int32)
    dupb = zero16
    dupc = zero16
    for c in range(_CHUNKS):
        kidx = idx_v[pl.ds(c * 16, 16)]
        klab = lab_v[b, pl.ds(c * 16, 16)]
        for i in range(16):
            k = c * 16 + i
            later = jnp.where(k > jpos, one16, zero16)
            same = later * jnp.where(kidx[i] == jidx, one16, zero16)
            dupb = dupb | same
            dupc = dupc | (same * jnp.where(klab[i] == jlab, one16, zero16))

    validm = jnp.where(jidx >= 0, one16, zero16)
    keepc = validm * (1 - dupc)
    winner = validm * (1 - dupb)

    ca = safe_cell & ~127
    descs = []
    for i in range(16):
        cai = pl.multiple_of(ca[i], 128)
        dc = pltpu.make_async_copy(
            cls_hbm.at[jlabc[i], :, pl.ds(cai, 128)], cwin_v.at[i], sem)
        db = pltpu.make_async_copy(
            boxp_hbm.at[:, :, pl.ds(cai, 128)], bwin_v.at[i], sem)
        pc = keepc[i] > 0
        pb = winner[i] > 0

        @pl.when(pc)
        def _(dc=dc):
            dc.start()

        @pl.when(pb)
        def _(db=db):
            db.start()

        descs.append((pc, dc))
        descs.append((pb, db))
    for pred, dsc in descs:
        @pl.when(pred)
        def _(dsc=dsc):
            dsc.wait()

    clow = safe_cell & 127
    zf16 = jnp.zeros((16,), jnp.float32)
    cvals = zf16
    bp = [zf16] * BOX_DIM
    for i in range(16):
        st = pl.multiple_of(clow[i] & ~15, 16)
        r = jnp.zeros((16,), jnp.int32) + (clow[i] & 15)
        csub = cwin_v[i, b, pl.ds(st, 16)]
        ev = _dyn_gather16(csub, r)
        cvals = jnp.where(iota16 == i, ev, cvals)
        for d in range(BOX_DIM):
            bsub = bwin_v[i, d, b, pl.ds(st, 16)]
            evd = _dyn_gather16(bsub, r)
            bp[d] = jnp.where(iota16 == i, evd, bp[d])

    sl1 = jnp.zeros((16,), jnp.float32)
    for d in range(BOX_DIM):
        diff = bp[d] - gtb_v[d, b, pl.ds(jlo, 16)]
        ad = jnp.abs(diff)
        sl1 = sl1 + jnp.where(ad < 1.0, 0.5 * diff * diff, ad - 0.5)

    stage_v[0, :] = jnp.where(keepc > 0, -cvals, 0.0)
    stage_v[1, :] = jnp.where(winner > 0, sl1, 0.0)
    stage_v[2, :] = jnp.where(winner > 0, 1.0, 0.0)
    pltpu.sync_copy(stage_v, out_hbm.at[wid])


def _sc_sparse_part(cls_t, boxp_t, gtb_t, lab, msk):
    mesh = plsc.VectorSubcoreMesh(core_axis_name="c", subcore_axis_name="s",
                                  num_cores=_NC, num_subcores=_NS)
    fn = pl.kernel(
        _sc_body,
        out_type=jax.ShapeDtypeStruct((_NW, 3, 16), jnp.float32),
        mesh=mesh,
        scratch_types=[
            pltpu.VMEM((BOX_DIM, BSZ, NB), jnp.float32),
            pltpu.VMEM((BSZ, NB), jnp.int32),
            pltpu.VMEM((BSZ, NB), jnp.float32),
            pltpu.VMEM((NB,), jnp.int32),
            pltpu.VMEM((16, BSZ, 128), jnp.float32),
            pltpu.VMEM((16, BOX_DIM, BSZ, 128), jnp.float32),
            pltpu.VMEM((3, 16), jnp.float32),
            pltpu.SemaphoreType.DMA,
        ],
    )
    return fn(cls_t, boxp_t, gtb_t, lab, msk)


def kernel(cls_logits, box_preds, gt_boxes, gt_labels, gt_masks):
    bsz = cls_logits.shape[0]
    cls_t = jnp.transpose(cls_logits, (2, 0, 1))
    boxp_t = jnp.transpose(box_preds, (2, 0, 1))
    gtb_t = jnp.transpose(gt_boxes, (2, 0, 1))
    lab = gt_labels.astype(jnp.int32)

    dense_sum = _dense_softplus_sum(cls_t)
    parts = _sc_sparse_part(cls_t, boxp_t, gtb_t, lab, gt_masks)
    sums = parts.sum(axis=(0, 2))

    cls_loss = (dense_sum + sums[0]) / (bsz * NUM_CELLS)
    box_loss = jnp.where(sums[2] > 0, sums[1] / (sums[2] + 1e-6), 0.0)
    total = cls_loss + box_loss
    return total, cls_loss, box_loss


# speedup vs baseline: 1.4713x; 1.0506x over previous
import functools

import jax
import jax.numpy as jnp
from jax import lax
from jax.experimental import pallas as pl
from jax.experimental.pallas import tpu as pltpu
from jax.experimental.pallas import tpu_sc as plsc

X_MIN, X_MAX = -50.0, 50.0
Y_MIN, Y_MAX = -50.0, 50.0
RES = 0.25
BEV_W = int((X_MAX - X_MIN) / RES)
BEV_H = int((Y_MAX - Y_MIN) / RES)
NUM_CELLS = BEV_W * BEV_H
NUM_CLASSES = 10
BOX_DIM = 7
BSZ = 4
NB = 128

_NC, _NS = 2, 16
_NW = _NC * _NS
_CHUNKS = NB // 16
assert BSZ * _CHUNKS == _NW


def _softplus_sum_body(x_ref, o_ref):
    x = x_ref[...]
    ax = jnp.abs(x)
    s = jnp.sum(0.5 * (x + ax) + jnp.log(1.0 + jnp.exp(-ax)))

    @pl.when(pl.program_id(0) == 0)
    def _():
        o_ref[0, 0] = 0.0

    o_ref[0, 0] += s


def _dense_softplus_sum(cls_t):
    out = pl.pallas_call(
        _softplus_sum_body,
        grid=(5,),
        in_specs=[pl.BlockSpec((NUM_CLASSES, BSZ, 32000), lambda i: (0, 0, i))],
        out_specs=pl.BlockSpec(memory_space=pltpu.SMEM),
        out_shape=jax.ShapeDtypeStruct((1, 1), jnp.float32),
    )(cls_t)
    return out[0, 0]


def _dyn_gather16(vec, idx):
    dn = lax.GatherDimensionNumbers(offset_dims=(), collapsed_slice_dims=(0,),
                                    start_index_map=(0,))
    return lax.gather(vec, idx[:, None], dn, (1,),
                      mode=lax.GatherScatterMode.PROMISE_IN_BOUNDS)


def _sc_body(cls_hbm, boxp_hbm, gtb_hbm, lab_hbm, msk_hbm, out_hbm,
             gtb_v, lab_v, msk_v, idx_v, cwin_v, bwin_v, stage_v,
             sem):
    cid = lax.axis_index("c")
    sid = lax.axis_index("s")
    wid = sid * _NC + cid
    b = wid // _CHUNKS
    jc = wid % _CHUNKS
    jlo = jc * 16

    pltpu.sync_copy(gtb_hbm, gtb_v)
    pltpu.sync_copy(lab_hbm, lab_v)
    pltpu.sync_copy(msk_hbm, msk_v)

    iota16 = lax.iota(jnp.int32, 16)

    for c in range(_CHUNKS):
        sl = pl.ds(c * 16, 16)
        x = gtb_v[0, b, sl]
        y = gtb_v[1, b, sl]
        m = msk_v[b, sl]
        lb = lab_v[b, sl]
        valid = ((m > 0.5) & (lb >= 0)
                 & (x >= X_MIN) & (x <= X_MAX)
                 & (y >= Y_MIN) & (y <= Y_MAX))
        gx = jnp.clip(((x - X_MIN) / RES).astype(jnp.int32), 0, BEV_W - 1)
        gy = jnp.clip(((y - Y_MIN) / RES).astype(jnp.int32), 0, BEV_H - 1)
        cell = gy * BEV_W + gx
        idx_v[pl.ds(c * 16, 16)] = jnp.where(valid, cell, -1)

    jpos = jlo + iota16
    jidx = idx_v[pl.ds(jlo, 16)]
    jlab = lab_v[b, pl.ds(jlo, 16)]
    safe_cell = jnp.where(jidx >= 0, jidx, 0)
    jlabc = jnp.clip(jlab, 0, NUM_CLASSES - 1)

    one16 = jnp.ones((16,), jnp.int32)
    zero16 = jnp.zeros((16,), jnp.int32)
    dupb = zero16
    dupc = zero16
    for c in range(_CHUNKS):
        kidx = idx_v[pl.ds(c * 16, 16)]
        klab = lab_v[b, pl.ds(c * 16, 16)]
        for i in range(16):
            k = c * 16 + i
            later = jnp.where(k > jpos, one16, zero16)
            same = later * jnp.where(kidx[i] == jidx, one16, zero16)
            dupb = dupb | same
            dupc = dupc | (same * jnp.where(klab[i] == jlab, one16, zero16))

    validm = jnp.where(jidx >= 0, one16, zero16)
    keepc = validm * (1 - dupc)
    winner = validm * (1 - dupb)

    ca = safe_cell & ~127
    descs = []
    for i in range(16):
        cai = pl.multiple_of(ca[i], 128)
        dc = pltpu.make_async_copy(
            cls_hbm.at[jlabc[i], :, pl.ds(cai, 128)], cwin_v.at[i], sem)
        db = pltpu.make_async_copy(
            boxp_hbm.at[:, :, pl.ds(cai, 128)], bwin_v.at[i], sem)
        pc = keepc[i] > 0
        pb = winner[i] > 0

        @pl.when(pc)
        def _(dc=dc):
            dc.start()

        @pl.when(pb)
        def _(db=db):
            db.start()

        descs.append((pc, dc))
        descs.append((pb, db))
    for pred, dsc in descs:
        @pl.when(pred)
        def _(dsc=dsc):
            dsc.wait()

    clow = safe_cell & 127
    zf16 = jnp.zeros((16,), jnp.float32)
    cvals = zf16
    bp = [zf16] * BOX_DIM
    for i in range(16):
        st = pl.multiple_of(clow[i] & ~15, 16)
        r = jnp.zeros((16,), jnp.int32) + (clow[i] & 15)
        csub = cwin_v[i, b, pl.ds(st, 16)]
        ev = _dyn_gather16(csub, r)
        cvals = jnp.where(iota16 == i, ev, cvals)
        for d in range(BOX_DIM):
            bsub = bwin_v[i, d, b, pl.ds(st, 16)]
            evd = _dyn_gather16(bsub, r)
            bp[d] = jnp.where(iota16 == i, evd, bp[d])

    sl1 = jnp.zeros((16,), jnp.float32)
    for d in range(BOX_DIM):
        diff = bp[d] - gtb_v[d, b, pl.ds(jlo, 16)]
        ad = jnp.abs(diff)
        sl1 = sl1 + jnp.where(ad < 1.0, 0.5 * diff * diff, ad - 0.5)

    stage_v[0, :] = jnp.where(keepc > 0, -cvals, 0.0)
    stage_v[1, :] = jnp.where(winner > 0, sl1, 0.0)
    stage_v[2, :] = jnp.where(winner > 0, 1.0, 0.0)
    pltpu.sync_copy(stage_v, out_hbm.at[wid])


def _sc_sparse_part(cls_t, boxp_t, gtb_t, lab, msk):
    mesh = plsc.VectorSubcoreMesh(core_axis_name="c", subcore_axis_name="s",
                                  num_cores=_NC, num_subcores=_NS)
    fn = pl.kernel(
        _sc_body,
        out_type=jax.ShapeDtypeStruct((_NW, 3, 16), jnp.float32),
        mesh=mesh,
        scratch_types=[
            pltpu.VMEM((BOX_DIM, BSZ, NB), jnp.float32),
            pltpu.VMEM((BSZ, NB), jnp.int32),
            pltpu.VMEM((BSZ, NB), jnp.float32),
            pltpu.VMEM((NB,), jnp.int32),
            pltpu.VMEM((16, BSZ, 128), jnp.float32),
            pltpu.VMEM((16, BOX_DIM, BSZ, 128), jnp.float32),
            pltpu.VMEM((3, 16), jnp.float32),
            pltpu.SemaphoreType.DMA,
        ],
    )
    return fn(cls_t, boxp_t, gtb_t, lab, msk)


def kernel(cls_logits, box_preds, gt_boxes, gt_labels, gt_masks):
    bsz = cls_logits.shape[0]
    cls_t = jnp.transpose(cls_logits, (2, 0, 1))
    boxp_t = jnp.transpose(box_preds, (2, 0, 1))
    gtb_t = jnp.transpose(gt_boxes, (2, 0, 1))
    lab = gt_labels.astype(jnp.int32)

    dense_sum = _dense_softplus_sum(cls_t)
    parts = _sc_sparse_part(cls_t, boxp_t, gtb_t, lab, gt_masks)
    sums = parts.sum(axis=(0, 2))

    cls_loss = (dense_sum + sums[0]) / (bsz * NUM_CELLS)
    box_loss = jnp.where(sums[2] > 0, sums[1] / (sums[2] + 1e-6), 0.0)
    total = cls_loss + box_loss
    return total, cls_loss, box_loss


# speedup vs baseline: 1.4828x; 1.0078x over previous
import functools

import jax
import jax.numpy as jnp
from jax import lax
from jax.experimental import pallas as pl
from jax.experimental.pallas import tpu as pltpu
from jax.experimental.pallas import tpu_sc as plsc

X_MIN, X_MAX = -50.0, 50.0
Y_MIN, Y_MAX = -50.0, 50.0
RES = 0.25
BEV_W = int((X_MAX - X_MIN) / RES)
BEV_H = int((Y_MAX - Y_MIN) / RES)
NUM_CELLS = BEV_W * BEV_H
NUM_CLASSES = 10
BOX_DIM = 7
BSZ = 4
NB = 128

_NC, _NS = 2, 16
_NW = _NC * _NS
_CHUNKS = NB // 16
assert BSZ * _CHUNKS == _NW


def _softplus_sum_body(x_ref, o_ref):
    x = x_ref[...]
    ax = jnp.abs(x)
    s = jnp.sum(0.5 * (x + ax) + jnp.log(1.0 + jnp.exp(-ax)))

    @pl.when(pl.program_id(0) == 0)
    def _():
        o_ref[0, 0] = 0.0

    o_ref[0, 0] += s


def _dense_softplus_sum(cls_t):
    out = pl.pallas_call(
        _softplus_sum_body,
        grid=(2,),
        in_specs=[pl.BlockSpec((NUM_CLASSES, BSZ, 80000), lambda i: (0, 0, i))],
        out_specs=pl.BlockSpec(memory_space=pltpu.SMEM),
        out_shape=jax.ShapeDtypeStruct((1, 1), jnp.float32),
    )(cls_t)
    return out[0, 0]


def _dyn_gather16(vec, idx):
    dn = lax.GatherDimensionNumbers(offset_dims=(), collapsed_slice_dims=(0,),
                                    start_index_map=(0,))
    return lax.gather(vec, idx[:, None], dn, (1,),
                      mode=lax.GatherScatterMode.PROMISE_IN_BOUNDS)


def _sc_body(cls_hbm, boxp_hbm, gtb_hbm, lab_hbm, msk_hbm, out_hbm,
             gtb_v, lab_v, msk_v, idx_v, cwin_v, bwin_v, stage_v,
             sem):
    cid = lax.axis_index("c")
    sid = lax.axis_index("s")
    wid = sid * _NC + cid
    b = wid // _CHUNKS
    jc = wid % _CHUNKS
    jlo = jc * 16

    pltpu.sync_copy(gtb_hbm, gtb_v)
    pltpu.sync_copy(lab_hbm, lab_v)
    pltpu.sync_copy(msk_hbm, msk_v)

    iota16 = lax.iota(jnp.int32, 16)

    for c in range(_CHUNKS):
        sl = pl.ds(c * 16, 16)
        x = gtb_v[0, b, sl]
        y = gtb_v[1, b, sl]
        m = msk_v[b, sl]
        lb = lab_v[b, sl]
        valid = ((m > 0.5) & (lb >= 0)
                 & (x >= X_MIN) & (x <= X_MAX)
                 & (y >= Y_MIN) & (y <= Y_MAX))
        gx = jnp.clip(((x - X_MIN) / RES).astype(jnp.int32), 0, BEV_W - 1)
        gy = jnp.clip(((y - Y_MIN) / RES).astype(jnp.int32), 0, BEV_H - 1)
        cell = gy * BEV_W + gx
        idx_v[pl.ds(c * 16, 16)] = jnp.where(valid, cell, -1)

    jpos = jlo + iota16
    jidx = idx_v[pl.ds(jlo, 16)]
    jlab = lab_v[b, pl.ds(jlo, 16)]
    safe_cell = jnp.where(jidx >= 0, jidx, 0)
    jlabc = jnp.clip(jlab, 0, NUM_CLASSES - 1)

    one16 = jnp.ones((16,), jnp.int32)
    zero16 = jnp.zeros((16,), jnp.int32)
    dupb = zero16
    dupc = zero16
    for c in range(_CHUNKS):
        kidx = idx_v[pl.ds(c * 16, 16)]
        klab = lab_v[b, pl.ds(c * 16, 16)]
        for i in range(16):
            k = c * 16 + i
            later = jnp.where(k > jpos, one16, zero16)
            same = later * jnp.where(kidx[i] == jidx, one16, zero16)
            dupb = dupb | same
            dupc = dupc | (same * jnp.where(klab[i] == jlab, one16, zero16))

    validm = jnp.where(jidx >= 0, one16, zero16)
    keepc = validm * (1 - dupc)
    winner = validm * (1 - dupb)

    ca = safe_cell & ~127
    descs = []
    for i in range(16):
        cai = pl.multiple_of(ca[i], 128)
        dc = pltpu.make_async_copy(
            cls_hbm.at[jlabc[i], :, pl.ds(cai, 128)], cwin_v.at[i], sem)
        db = pltpu.make_async_copy(
            boxp_hbm.at[:, :, pl.ds(cai, 128)], bwin_v.at[i], sem)
        pc = keepc[i] > 0
        pb = winner[i] > 0

        @pl.when(pc)
        def _(dc=dc):
            dc.start()

        @pl.when(pb)
        def _(db=db):
            db.start()

        descs.append((pc, dc))
        descs.append((pb, db))
    for pred, dsc in descs:
        @pl.when(pred)
        def _(dsc=dsc):
            dsc.wait()

    clow = safe_cell & 127
    zf16 = jnp.zeros((16,), jnp.float32)
    cvals = zf16
    bp = [zf16] * BOX_DIM
    for i in range(16):
        st = pl.multiple_of(clow[i] & ~15, 16)
        r = jnp.zeros((16,), jnp.int32) + (clow[i] & 15)
        csub = cwin_v[i, b, pl.ds(st, 16)]
        ev = _dyn_gather16(csub, r)
        cvals = jnp.where(iota16 == i, ev, cvals)
        for d in range(BOX_DIM):
            bsub = bwin_v[i, d, b, pl.ds(st, 16)]
            evd = _dyn_gather16(bsub, r)
            bp[d] = jnp.where(iota16 == i, evd, bp[d])

    sl1 = jnp.zeros((16,), jnp.float32)
    for d in range(BOX_DIM):
        diff = bp[d] - gtb_v[d, b, pl.ds(jlo, 16)]
        ad = jnp.abs(diff)
        sl1 = sl1 + jnp.where(ad < 1.0, 0.5 * diff * diff, ad - 0.5)

    stage_v[0, :] = jnp.where(keepc > 0, -cvals, 0.0)
    stage_v[1, :] = jnp.where(winner > 0, sl1, 0.0)
    stage_v[2, :] = jnp.where(winner > 0, 1.0, 0.0)
    pltpu.sync_copy(stage_v, out_hbm.at[wid])


def _sc_sparse_part(cls_t, boxp_t, gtb_t, lab, msk):
    mesh = plsc.VectorSubcoreMesh(core_axis_name="c", subcore_axis_name="s",
                                  num_cores=_NC, num_subcores=_NS)
    fn = pl.kernel(
        _sc_body,
        out_type=jax.ShapeDtypeStruct((_NW, 3, 16), jnp.float32),
        mesh=mesh,
        scratch_types=[
            pltpu.VMEM((BOX_DIM, BSZ, NB), jnp.float32),
            pltpu.VMEM((BSZ, NB), jnp.int32),
            pltpu.VMEM((BSZ, NB), jnp.float32),
            pltpu.VMEM((NB,), jnp.int32),
            pltpu.VMEM((16, BSZ, 128), jnp.float32),
            pltpu.VMEM((16, BOX_DIM, BSZ, 128), jnp.float32),
            pltpu.VMEM((3, 16), jnp.float32),
            pltpu.SemaphoreType.DMA,
        ],
    )
    return fn(cls_t, boxp_t, gtb_t, lab, msk)


def kernel(cls_logits, box_preds, gt_boxes, gt_labels, gt_masks):
    bsz = cls_logits.shape[0]
    cls_t = jnp.transpose(cls_logits, (2, 0, 1))
    boxp_t = jnp.transpose(box_preds, (2, 0, 1))
    gtb_t = jnp.transpose(gt_boxes, (2, 0, 1))
    lab = gt_labels.astype(jnp.int32)

    dense_sum = _dense_softplus_sum(cls_t)
    parts = _sc_sparse_part(cls_t, boxp_t, gtb_t, lab, gt_masks)
    sums = parts.sum(axis=(0, 2))

    cls_loss = (dense_sum + sums[0]) / (bsz * NUM_CELLS)
    box_loss = jnp.where(sums[2] > 0, sums[1] / (sums[2] + 1e-6), 0.0)
    total = cls_loss + box_loss
    return total, cls_loss, box_loss
